# baseline (speedup 1.0000x reference)
import functools

import jax
import jax.numpy as jnp
import numpy as np
from jax import lax
from jax.experimental import pallas as pl
from jax.experimental.pallas import tpu as pltpu
from jax.experimental.pallas import tpu_sc as plsc

N = 10000
E = 320000
D_IN = 128
D_OUT = 128
HALF = 64
NC = 2
NT = 16
LANES = 16
NBUF = 3
CHUNK = 256
SUB = 128
GSUB = CHUNK // SUB
EDGES_PER_TILE = E // NT
NCH = 79
LAST_BASE = EDGES_PER_TILE - CHUNK
OVERLAP = NCH * CHUNK - EDGES_PER_TILE
ROWS_A = 624
ROWS_B = 640

_PERM = np.empty((D_OUT,), np.int32)
for _g in range(D_OUT // 32):
    for _i in range(16):
        _PERM[_g * 32 + 2 * _i] = _g * 32 + _i
        _PERM[_g * 32 + 2 * _i + 1] = _g * 32 + 16 + _i


def _matmul_body(x_ref, w_ref, o_ref):
    o_ref[...] = jnp.dot(x_ref[...], w_ref[...],
                         preferred_element_type=jnp.float32
                         ).astype(jnp.bfloat16)


def _matmul(x, W):
    BLK = 2000
    return pl.pallas_call(
        _matmul_body,
        grid=(N // BLK,),
        in_specs=[
            pl.BlockSpec((BLK, D_IN), lambda i: (i, 0)),
            pl.BlockSpec((D_IN, D_OUT), lambda i: (0, 0)),
        ],
        out_specs=pl.BlockSpec((BLK, D_OUT), lambda i: (i, 0)),
        out_shape=jax.ShapeDtypeStruct((N, D_OUT), jnp.bfloat16),
    )(x, W)


def _spmm_sc(m2, adj_index, adj_values, b2):
    mesh = plsc.VectorSubcoreMesh(core_axis_name="c", subcore_axis_name="s")

    @functools.partial(
        pl.kernel,
        out_type=jax.ShapeDtypeStruct((N, D_OUT), jnp.float32),
        mesh=mesh,
        compiler_params=pltpu.CompilerParams(use_tc_tiling_on_sc=False,
                                             needs_layout_passes=False),
        scratch_types=[
            pltpu.VMEM((NBUF, CHUNK), jnp.int32),
            pltpu.VMEM((NBUF, CHUNK), jnp.int32),
            pltpu.VMEM((NBUF, GSUB, SUB), jnp.int32),
            pltpu.VMEM((NBUF, CHUNK), jnp.float32),
            pltpu.VMEM((NBUF, CHUNK, HALF), jnp.bfloat16),
            pltpu.VMEM((NBUF, CHUNK, HALF), jnp.float32),
            pltpu.VMEM((HALF,), jnp.float32),
            pltpu.VMEM_SHARED((N, HALF), jnp.float32),
            [pltpu.SemaphoreType.DMA] * NBUF,
            [pltpu.SemaphoreType.DMA] * NBUF,
            [pltpu.SemaphoreType.DMA] * NBUF,
        ],
    )
    def k(m_hbm, adj_hbm, val_hbm, b_hbm, out_hbm,
          srcv, dstv, dst2, valv, rows, scaled, bb, acc,
          sem_g, sem_i, sem_s):
        c = lax.axis_index("c")
        s = lax.axis_index("s")

        pltpu.sync_copy(b_hbm.at[c], bb)
        bvec = [bb[pl.ds(j * LANES, LANES)] for j in range(HALF // LANES)]
        for t in range(NBUF):
            @plsc.parallel_loop(0, CHUNK, unroll=4)
            def _(i):
                for j in range(HALF // LANES):
                    scaled[t, i, pl.ds(j * LANES, LANES)] = bvec[j]

        @pl.when(s < NT - 1)
        def _():
            base = s * ROWS_A
            pltpu.sync_copy(scaled.at[0], acc.at[pl.ds(base, CHUNK)])
            pltpu.sync_copy(scaled.at[1],
                            acc.at[pl.ds(base + CHUNK, CHUNK)])
            pltpu.sync_copy(scaled.at[2, pl.ds(0, ROWS_A - 2 * CHUNK)],
                            acc.at[pl.ds(base + 2 * CHUNK,
                                         ROWS_A - 2 * CHUNK)])

        @pl.when(s == NT - 1)
        def _():
            base = (NT - 1) * ROWS_A
            pltpu.sync_copy(scaled.at[0], acc.at[pl.ds(base, CHUNK)])
            pltpu.sync_copy(scaled.at[1],
                            acc.at[pl.ds(base + CHUNK, CHUNK)])
            pltpu.sync_copy(scaled.at[2, pl.ds(0, ROWS_B - 2 * CHUNK)],
                            acc.at[pl.ds(base + 2 * CHUNK,
                                         ROWS_B - 2 * CHUNK)])
        plsc.subcore_barrier()

        base0 = s * EDGES_PER_TILE
        cvec = jnp.full((LANES,), 1, jnp.int32) * c

        def fire_idx(kk, p):
            base = base0 + jnp.minimum(kk * CHUNK, LAST_BASE)
            pltpu.async_copy(adj_hbm.at[1, pl.ds(base, CHUNK)], srcv.at[p],
                             sem_i[p])
            pltpu.async_copy(adj_hbm.at[0, pl.ds(base, CHUNK)], dstv.at[p],
                             sem_i[p])
            pltpu.async_copy(val_hbm.at[pl.ds(base, CHUNK)], valv.at[p],
                             sem_i[p])

        def wait_idx(p):
            pltpu.make_async_copy(adj_hbm.at[1, pl.ds(0, CHUNK)], srcv.at[p],
                                  sem_i[p]).wait()
            pltpu.make_async_copy(adj_hbm.at[0, pl.ds(0, CHUNK)], dstv.at[p],
                                  sem_i[p]).wait()
            pltpu.make_async_copy(val_hbm.at[pl.ds(0, CHUNK)], valv.at[p],
                                  sem_i[p]).wait()

        def zero_overlap(p):
            @plsc.parallel_loop(0, OVERLAP // LANES, unroll=4)
            def _(i):
                valv[p, pl.ds(i * LANES, LANES)] = jnp.zeros((LANES,),
                                                             jnp.float32)

        def build_idx(p):
            @plsc.parallel_loop(0, CHUNK // LANES, unroll=4)
            def _(i):
                g = i // (SUB // LANES)
                r = i % (SUB // LANES)
                dst2[p, g, pl.ds(r * LANES, LANES)] = \
                    dstv[p, pl.ds(i * LANES, LANES)]

            @plsc.parallel_loop(0, CHUNK // LANES, unroll=4)
            def _(i):
                sl = pl.ds(i * LANES, LANES)
                srcv[p, sl] = (srcv[p, sl] << 1) + cvec

        def fire_gather(p):
            for g in range(GSUB):
                pltpu.async_copy(m_hbm.at[srcv.at[p, pl.ds(g * SUB, SUB)]],
                                 rows.at[p, pl.ds(g * SUB, SUB)], sem_g[p])

        def wait_gather(p):
            pltpu.make_async_copy(m_hbm.at[pl.ds(0, CHUNK)], rows.at[p],
                                  sem_g[p]).wait()

        def fire_scatter(p):
            for g in range(GSUB):
                pltpu.async_copy(scaled.at[p, pl.ds(g * SUB, SUB)],
                                 acc.at[dst2.at[p, g]], sem_s[p], add=True)

        def wait_scatter(p):
            pltpu.make_async_copy(out_hbm.at[pl.ds(0, CHUNK), pl.ds(0, HALF)],
                                  scaled.at[p], sem_s[p]).wait()

        def scale(p):
            @plsc.parallel_loop(0, CHUNK // LANES, unroll=4)
            def _(i):
                vv = valv[p, pl.ds(i * LANES, LANES)]
                for l in range(LANES):
                    r = i * LANES + l
                    v = vv[l]
                    for h in range(2):
                        x2 = rows[p, r, pl.ds(h * 2 * LANES, 2 * LANES)]
                        a, b_ = plsc.unpack(
                            x2, format=plsc.PackFormat.INTERLEAVED)
                        scaled[p, r, pl.ds(h * 2 * LANES, LANES)] = a * v
                        scaled[p, r, pl.ds((h * 2 + 1) * LANES, LANES)] = \
                            b_ * v

        fire_idx(0, 0)
        wait_idx(0)
        build_idx(0)
        fire_gather(0)
        fire_idx(1, 1)

        def half_step(kk, p):
            p1 = (p + 1) % NBUF
            p2 = (p + 2) % NBUF

            @pl.when(kk + 1 < NCH)
            def _():
                wait_idx(p1)

                @pl.when(kk + 1 == NCH - 1)
                def _():
                    zero_overlap(p1)

            @pl.when(kk >= 2)
            def _():
                wait_scatter(p1)

            @pl.when(kk + 1 < NCH)
            def _():
                build_idx(p1)
                fire_gather(p1)

            wait_gather(p)
            scale(p)
            fire_scatter(p)

            @pl.when(kk + 2 < NCH)
            def _():
                fire_idx(kk + 2, p2)

        def triple_body(j, carry):
            half_step(3 * j, 0)
            half_step(3 * j + 1, 1)
            half_step(3 * j + 2, 2)
            return carry
        lax.fori_loop(0, NCH // NBUF, triple_body, 0)
        for kk in range(NBUF * (NCH // NBUF), NCH):
            half_step(kk, kk % NBUF)

        wait_scatter((NCH - 2) % NBUF)
        wait_scatter((NCH - 1) % NBUF)

        plsc.subcore_barrier()

        @pl.when(s < NT - 1)
        def _():
            pltpu.sync_copy(
                acc.at[pl.ds(s * ROWS_A, ROWS_A)],
                out_hbm.at[pl.ds(s * ROWS_A, ROWS_A), pl.ds(c * HALF, HALF)])

        @pl.when(s == NT - 1)
        def _():
            pltpu.sync_copy(
                acc.at[pl.ds((NT - 1) * ROWS_A, ROWS_B)],
                out_hbm.at[pl.ds((NT - 1) * ROWS_A, ROWS_B),
                           pl.ds(c * HALF, HALF)])

    return k(m2, adj_index, adj_values, b2)


def kernel(x, adj_index, adj_values, W, b):
    m2 = _matmul(x, W[:, _PERM]).reshape(NC * N, HALF)
    b2 = b.reshape(NC, HALF)
    return _spmm_sc(m2, adj_index, adj_values, b2)


# speedup vs baseline: 1.0225x; 1.0225x over previous
import functools

import jax
import jax.numpy as jnp
from jax import lax
from jax.experimental import pallas as pl
from jax.experimental.pallas import tpu as pltpu
from jax.experimental.pallas import tpu_sc as plsc

N = 10000
E = 320000
D_IN = 128
D_OUT = 128
HALF = 64
NC = 2
NT = 16
LANES = 16
NBUF = 3
CHUNK = 384
SUB = 128
GSUB = CHUNK // SUB
EDGES_PER_TILE = E // NT
NCH = 53
LAST_BASE = EDGES_PER_TILE - CHUNK
OVERLAP = NCH * CHUNK - EDGES_PER_TILE
ROWS_A = 624
ROWS_B = 640


def _matmul_body(x_ref, w_ref, o_ref):
    o_ref[...] = jnp.dot(x_ref[...], w_ref[...],
                         preferred_element_type=jnp.float32)


def _matmul(x, W):
    BLK = 2000
    return pl.pallas_call(
        _matmul_body,
        grid=(N // BLK,),
        in_specs=[
            pl.BlockSpec((BLK, D_IN), lambda i: (i, 0)),
            pl.BlockSpec((D_IN, D_OUT), lambda i: (0, 0)),
        ],
        out_specs=pl.BlockSpec((BLK, D_OUT), lambda i: (i, 0)),
        out_shape=jax.ShapeDtypeStruct((N, D_OUT), jnp.float32),
    )(x, W)


def _spmm_sc(m2, adj_index, adj_values, b2):
    mesh = plsc.VectorSubcoreMesh(core_axis_name="c", subcore_axis_name="s")

    @functools.partial(
        pl.kernel,
        out_type=jax.ShapeDtypeStruct((N, D_OUT), jnp.float32),
        mesh=mesh,
        compiler_params=pltpu.CompilerParams(use_tc_tiling_on_sc=False),
        scratch_types=[
            pltpu.VMEM((NBUF, CHUNK), jnp.int32),
            pltpu.VMEM((NBUF, CHUNK), jnp.int32),
            pltpu.VMEM((NBUF, CHUNK), jnp.float32),
            pltpu.VMEM((NBUF, CHUNK, HALF), jnp.float32),
            pltpu.VMEM((HALF,), jnp.float32),
            pltpu.VMEM_SHARED((N, HALF), jnp.float32),
            [pltpu.SemaphoreType.DMA] * NBUF,
            [pltpu.SemaphoreType.DMA] * NBUF,
            [pltpu.SemaphoreType.DMA] * NBUF,
        ],
    )
    def k(m_hbm, adj_hbm, val_hbm, b_hbm, out_hbm,
          srcv, dstv, valv, rows, bb, acc,
          sem_g, sem_i, sem_s):
        c = lax.axis_index("c")
        s = lax.axis_index("s")

        pltpu.sync_copy(b_hbm.at[c], bb)
        bvec = [bb[pl.ds(j * LANES, LANES)] for j in range(HALF // LANES)]

        @plsc.parallel_loop(0, CHUNK, unroll=4)
        def _(i):
            for j in range(HALF // LANES):
                rows[0, i, pl.ds(j * LANES, LANES)] = bvec[j]

        @plsc.parallel_loop(0, ROWS_B - CHUNK, unroll=4)
        def _(i):
            for j in range(HALF // LANES):
                rows[1, i, pl.ds(j * LANES, LANES)] = bvec[j]

        @pl.when(s < NT - 1)
        def _():
            pltpu.sync_copy(rows.at[0], acc.at[pl.ds(s * ROWS_A, CHUNK)])
            pltpu.sync_copy(rows.at[1, pl.ds(0, ROWS_A - CHUNK)],
                            acc.at[pl.ds(s * ROWS_A + CHUNK, ROWS_A - CHUNK)])

        @pl.when(s == NT - 1)
        def _():
            base = (NT - 1) * ROWS_A
            pltpu.sync_copy(rows.at[0], acc.at[pl.ds(base, CHUNK)])
            pltpu.sync_copy(rows.at[1, pl.ds(0, ROWS_B - CHUNK)],
                            acc.at[pl.ds(base + CHUNK, ROWS_B - CHUNK)])
        plsc.subcore_barrier()

        base0 = s * EDGES_PER_TILE
        cvec = jnp.full((LANES,), 1, jnp.int32) * c

        def fire_idx(kk, p):
            base = base0 + jnp.minimum(kk * CHUNK, LAST_BASE)
            pltpu.async_copy(adj_hbm.at[1, pl.ds(base, CHUNK)], srcv.at[p],
                             sem_i[p])
            pltpu.async_copy(adj_hbm.at[0, pl.ds(base, CHUNK)], dstv.at[p],
                             sem_i[p])
            pltpu.async_copy(val_hbm.at[pl.ds(base, CHUNK)], valv.at[p],
                             sem_i[p])

        def wait_idx(p):
            pltpu.make_async_copy(adj_hbm.at[1, pl.ds(0, CHUNK)], srcv.at[p],
                                  sem_i[p]).wait()
            pltpu.make_async_copy(adj_hbm.at[0, pl.ds(0, CHUNK)], dstv.at[p],
                                  sem_i[p]).wait()
            pltpu.make_async_copy(val_hbm.at[pl.ds(0, CHUNK)], valv.at[p],
                                  sem_i[p]).wait()

        def zero_overlap(p):
            @plsc.parallel_loop(0, OVERLAP // LANES, unroll=4)
            def _(i):
                valv[p, pl.ds(i * LANES, LANES)] = jnp.zeros((LANES,),
                                                             jnp.float32)

        def build_idx(p):
            @plsc.parallel_loop(0, CHUNK // LANES, unroll=4)
            def _(i):
                sl = pl.ds(i * LANES, LANES)
                srcv[p, sl] = (srcv[p, sl] << 1) + cvec

        def fire_gather(p):
            for g in range(GSUB):
                pltpu.async_copy(m_hbm.at[srcv.at[p, pl.ds(g * SUB, SUB)]],
                                 rows.at[p, pl.ds(g * SUB, SUB)], sem_g[p])

        def wait_gather(p):
            pltpu.make_async_copy(m_hbm.at[pl.ds(0, CHUNK)], rows.at[p],
                                  sem_g[p]).wait()

        def fire_scatter(p):
            for g in range(GSUB):
                pltpu.async_copy(rows.at[p, pl.ds(g * SUB, SUB)],
                                 acc.at[dstv.at[p, pl.ds(g * SUB, SUB)]],
                                 sem_s[p], add=True)

        def wait_scatter(p):
            pltpu.make_async_copy(m_hbm.at[pl.ds(0, CHUNK)], rows.at[p],
                                  sem_s[p]).wait()

        def scale(p):
            @plsc.parallel_loop(0, CHUNK // LANES, unroll=8)
            def _(i):
                vv = valv[p, pl.ds(i * LANES, LANES)]
                for l in range(LANES):
                    r = i * LANES + l
                    v = vv[l]
                    for j in range(HALF // LANES):
                        sl = pl.ds(j * LANES, LANES)
                        rows[p, r, sl] = rows[p, r, sl] * v

        fire_idx(0, 0)
        wait_idx(0)
        build_idx(0)
        fire_gather(0)
        fire_idx(1, 1)

        def half_step(kk, p):
            p1 = (p + 1) % NBUF
            p2 = (p + 2) % NBUF

            @pl.when(kk + 1 < NCH)
            def _():
                wait_idx(p1)

                @pl.when(kk + 1 == NCH - 1)
                def _():
                    zero_overlap(p1)

            @pl.when(kk >= 2)
            def _():
                wait_scatter(p1)

            @pl.when(kk + 1 < NCH)
            def _():
                build_idx(p1)
                fire_gather(p1)

            wait_gather(p)
            scale(p)
            fire_scatter(p)

            @pl.when(kk + 2 < NCH)
            def _():
                fire_idx(kk + 2, p2)

        def triple_body(j, carry):
            half_step(3 * j, 0)
            half_step(3 * j + 1, 1)
            half_step(3 * j + 2, 2)
            return carry
        lax.fori_loop(0, NCH // NBUF, triple_body, 0)
        for kk in range(NBUF * (NCH // NBUF), NCH):
            half_step(kk, kk % NBUF)

        wait_scatter((NCH - 2) % NBUF)
        wait_scatter((NCH - 1) % NBUF)

        plsc.subcore_barrier()

        @pl.when(s < NT - 1)
        def _():
            pltpu.sync_copy(
                acc.at[pl.ds(s * ROWS_A, ROWS_A)],
                out_hbm.at[pl.ds(s * ROWS_A, ROWS_A), pl.ds(c * HALF, HALF)])

        @pl.when(s == NT - 1)
        def _():
            pltpu.sync_copy(
                acc.at[pl.ds((NT - 1) * ROWS_A, ROWS_B)],
                out_hbm.at[pl.ds((NT - 1) * ROWS_A, ROWS_B),
                           pl.ds(c * HALF, HALF)])

    return k(m2, adj_index, adj_values, b2)


def kernel(x, adj_index, adj_values, W, b):
    m2 = _matmul(x, W).reshape(NC * N, HALF)
    b2 = b.reshape(NC, HALF)
    return _spmm_sc(m2, adj_index, adj_values, b2)


# speedup vs baseline: 1.0444x; 1.0215x over previous
import functools

import jax
import jax.numpy as jnp
from jax import lax
from jax.experimental import pallas as pl
from jax.experimental.pallas import tpu as pltpu
from jax.experimental.pallas import tpu_sc as plsc

N = 10000
E = 320000
D_IN = 128
D_OUT = 128
HALF = 64
NC = 2
NT = 16
LANES = 16
NBUF = 3
CHUNK = 384
SUB = 128
GSUB = CHUNK // SUB
EDGES_PER_TILE = E // NT
NCH = 53
LAST_BASE = EDGES_PER_TILE - CHUNK
OVERLAP = NCH * CHUNK - EDGES_PER_TILE
ROWS_A = 624
ROWS_B = 640


def _matmul_body(x_ref, w_ref, o_ref):
    o_ref[...] = jnp.dot(x_ref[...], w_ref[...],
                         preferred_element_type=jnp.float32)


def _matmul(x, W):
    BLK = 2000
    return pl.pallas_call(
        _matmul_body,
        grid=(N // BLK,),
        in_specs=[
            pl.BlockSpec((BLK, D_IN), lambda i: (i, 0)),
            pl.BlockSpec((D_IN, D_OUT), lambda i: (0, 0)),
        ],
        out_specs=pl.BlockSpec((BLK, D_OUT), lambda i: (i, 0)),
        out_shape=jax.ShapeDtypeStruct((N, D_OUT), jnp.float32),
    )(x, W)


def _spmm_sc(m2, adj_index, adj_values, b2):
    mesh = plsc.VectorSubcoreMesh(core_axis_name="c", subcore_axis_name="s")

    @functools.partial(
        pl.kernel,
        out_type=jax.ShapeDtypeStruct((N, D_OUT), jnp.float32),
        mesh=mesh,
        compiler_params=pltpu.CompilerParams(use_tc_tiling_on_sc=False),
        scratch_types=[
            pltpu.VMEM((NBUF, CHUNK), jnp.int32),
            pltpu.VMEM((NBUF, CHUNK), jnp.int32),
            pltpu.VMEM((NBUF, CHUNK), jnp.float32),
            pltpu.VMEM((NBUF, CHUNK, HALF), jnp.float32),
            pltpu.VMEM((HALF,), jnp.float32),
            pltpu.VMEM_SHARED((N, HALF), jnp.float32),
            [pltpu.SemaphoreType.DMA] * NBUF,
            [pltpu.SemaphoreType.DMA] * NBUF,
            [pltpu.SemaphoreType.DMA] * NBUF,
        ],
    )
    def k(m_hbm, adj_hbm, val_hbm, b_hbm, out_hbm,
          srcv, dstv, valv, rows, bb, acc,
          sem_g, sem_i, sem_s):
        c = lax.axis_index("c")
        s = lax.axis_index("s")

        pltpu.sync_copy(b_hbm.at[c], bb)
        bvec = [bb[pl.ds(j * LANES, LANES)] for j in range(HALF // LANES)]

        @plsc.parallel_loop(0, CHUNK, unroll=4)
        def _(i):
            for j in range(HALF // LANES):
                rows[0, i, pl.ds(j * LANES, LANES)] = bvec[j]

        @plsc.parallel_loop(0, ROWS_B - CHUNK, unroll=4)
        def _(i):
            for j in range(HALF // LANES):
                rows[1, i, pl.ds(j * LANES, LANES)] = bvec[j]

        @pl.when(s < NT - 1)
        def _():
            pltpu.sync_copy(rows.at[0], acc.at[pl.ds(s * ROWS_A, CHUNK)])
            pltpu.sync_copy(rows.at[1, pl.ds(0, ROWS_A - CHUNK)],
                            acc.at[pl.ds(s * ROWS_A + CHUNK, ROWS_A - CHUNK)])

        @pl.when(s == NT - 1)
        def _():
            base = (NT - 1) * ROWS_A
            pltpu.sync_copy(rows.at[0], acc.at[pl.ds(base, CHUNK)])
            pltpu.sync_copy(rows.at[1, pl.ds(0, ROWS_B - CHUNK)],
                            acc.at[pl.ds(base + CHUNK, ROWS_B - CHUNK)])
        plsc.subcore_barrier()

        base0 = s * EDGES_PER_TILE
        cvec = jnp.full((LANES,), 1, jnp.int32) * c

        def fire_idx(kk, p):
            base = base0 + jnp.minimum(kk * CHUNK, LAST_BASE)
            pltpu.async_copy(adj_hbm.at[1, pl.ds(base, CHUNK)], srcv.at[p],
                             sem_i[p])
            pltpu.async_copy(adj_hbm.at[0, pl.ds(base, CHUNK)], dstv.at[p],
                             sem_i[p])
            pltpu.async_copy(val_hbm.at[pl.ds(base, CHUNK)], valv.at[p],
                             sem_i[p])

        def wait_idx(p):
            pltpu.make_async_copy(adj_hbm.at[1, pl.ds(0, CHUNK)], srcv.at[p],
                                  sem_i[p]).wait()
            pltpu.make_async_copy(adj_hbm.at[0, pl.ds(0, CHUNK)], dstv.at[p],
                                  sem_i[p]).wait()
            pltpu.make_async_copy(val_hbm.at[pl.ds(0, CHUNK)], valv.at[p],
                                  sem_i[p]).wait()

        def zero_overlap(p):
            @plsc.parallel_loop(0, OVERLAP // LANES, unroll=4)
            def _(i):
                valv[p, pl.ds(i * LANES, LANES)] = jnp.zeros((LANES,),
                                                             jnp.float32)

        def build_idx(p):
            @plsc.parallel_loop(0, CHUNK // LANES, unroll=4)
            def _(i):
                sl = pl.ds(i * LANES, LANES)
                srcv[p, sl] = (srcv[p, sl] << 1) + cvec

        def fire_gather(p):
            for g in range(GSUB):
                pltpu.async_copy(m_hbm.at[srcv.at[p, pl.ds(g * SUB, SUB)]],
                                 rows.at[p, pl.ds(g * SUB, SUB)], sem_g[p])

        def wait_gather(p):
            pltpu.make_async_copy(m_hbm.at[pl.ds(0, CHUNK)], rows.at[p],
                                  sem_g[p]).wait()

        def fire_scatter(p):
            for g in range(GSUB):
                pltpu.async_copy(rows.at[p, pl.ds(g * SUB, SUB)],
                                 acc.at[dstv.at[p, pl.ds(g * SUB, SUB)]],
                                 sem_s[p], add=True)

        def wait_scatter(p):
            pltpu.make_async_copy(m_hbm.at[pl.ds(0, CHUNK)], rows.at[p],
                                  sem_s[p]).wait()

        def scale(p):
            @plsc.parallel_loop(0, CHUNK // LANES, unroll=4)
            def _(i):
                vv = valv[p, pl.ds(i * LANES, LANES)]
                for l in range(LANES):
                    r = i * LANES + l
                    v = vv[l]
                    for j in range(HALF // LANES):
                        sl = pl.ds(j * LANES, LANES)
                        rows[p, r, sl] = rows[p, r, sl] * v

        fire_idx(0, 0)
        wait_idx(0)
        build_idx(0)
        fire_gather(0)
        fire_idx(1, 1)

        def half_step(kk, p):
            p1 = (p + 1) % NBUF
            p2 = (p + 2) % NBUF

            @pl.when(kk + 1 < NCH)
            def _():
                wait_idx(p1)

                @pl.when(kk + 1 == NCH - 1)
                def _():
                    zero_overlap(p1)

            @pl.when(kk >= 2)
            def _():
                wait_scatter(p1)

            @pl.when(kk + 1 < NCH)
            def _():
                build_idx(p1)
                fire_gather(p1)

            wait_gather(p)
            scale(p)
            fire_scatter(p)

            @pl.when(kk + 2 < NCH)
            def _():
                fire_idx(kk + 2, p2)

        def triple_body(j, carry):
            half_step(3 * j, 0)
            half_step(3 * j + 1, 1)
            half_step(3 * j + 2, 2)
            return carry
        lax.fori_loop(0, NCH // NBUF, triple_body, 0)
        for kk in range(NBUF * (NCH // NBUF), NCH):
            half_step(kk, kk % NBUF)

        wait_scatter((NCH - 2) % NBUF)
        wait_scatter((NCH - 1) % NBUF)

        plsc.subcore_barrier()

        @pl.when(s < NT - 1)
        def _():
            pltpu.sync_copy(
                acc.at[pl.ds(s * ROWS_A, ROWS_A)],
                out_hbm.at[pl.ds(s * ROWS_A, ROWS_A), pl.ds(c * HALF, HALF)])

        @pl.when(s == NT - 1)
        def _():
            pltpu.sync_copy(
                acc.at[pl.ds((NT - 1) * ROWS_A, ROWS_B)],
                out_hbm.at[pl.ds((NT - 1) * ROWS_A, ROWS_B),
                           pl.ds(c * HALF, HALF)])

    return k(m2, adj_index, adj_values, b2)


def kernel(x, adj_index, adj_values, W, b):
    m2 = _matmul(x, W).reshape(NC * N, HALF)
    b2 = b.reshape(NC, HALF)
    return _spmm_sc(m2, adj_index, adj_values, b2)
